# Initial kernel scaffold; baseline (speedup 1.0000x reference)
import jax
import jax.numpy as jnp
from jax.experimental import pallas as pl


def kernel(h_x, sp_x, evt_x, h_of, h_ox, planar_edge_index, nexus_src, nexus_dst, sp_evt_src, sp_evt_dst, params):
    raise NotImplementedError("write your pallas kernel here")


# speedup vs baseline: 3.2899x; 3.2899x over previous
import jax
import jax.numpy as jnp
from jax import lax
from jax.experimental import pallas as pl
from jax.experimental.pallas import tpu as pltpu
from jax.experimental.pallas import tpu_sc as plsc

_NS = 16
_TILE = 256
_ACC_LIMIT = 1_660_000
_EBUF = 1024


def _round_up(x, m):
    return -(-x // m) * m


def _half_rows(n):
    return _round_up(-(-n // 2), _TILE)


def _num_slices(half_p, S):
    k = 1
    while (half_p // k + 16) * 2 * S * 4 > _ACC_LIMIT:
        k *= 2
    return k


def _sc_edge_phase(x128, adst, bsrc, src_e, dst_e, S, half_p, n_slices):
    n_src_pad = x128.shape[0]
    E_pad = src_e.shape[0]
    E_per = E_pad // _NS
    slice_rows = half_p // n_slices
    nzc = slice_rows // 16
    nzi = -(-nzc // _NS)
    n_fb = S // 16
    C2 = 2 * S
    n_acc = C2 // 128
    n_eb = E_per // _EBUF
    e_rem = E_per - n_eb * _EBUF
    nch_cap = E_per // 16 + 1

    mesh = plsc.VectorSubcoreMesh(core_axis_name="c", subcore_axis_name="s",
                                  num_cores=2, num_subcores=_NS)

    def body(x_hbm, adst_hbm, bsrc_hbm, src_hbm, dst_hbm, *rest):
        outs_hbm = rest[:n_acc]
        (cdst_v, csrc_v, ebuf_d, ebuf_s, adsl_v, bsrc_v,
         rowbuf, zbuf, didxb) = rest[n_acc:n_acc + 9]
        sbufs = rest[n_acc + 9:n_acc + 9 + n_acc]
        accs = rest[n_acc + 9 + n_acc:]
        cid = lax.axis_index("c")
        sid = lax.axis_index("s")
        ebase = sid * E_per
        pltpu.sync_copy(bsrc_hbm, bsrc_v)
        zeros_f = jnp.zeros((16,), jnp.float32)
        for r in range(16):
            for fb in range(8):
                zbuf[r, pl.ds(fb * 16, 16)] = zeros_f

        def slice_body(sl, _):
            lo = cid * half_p + sl * slice_rows
            hi = lo + slice_rows
            pltpu.sync_copy(adst_hbm.at[pl.ds(lo, slice_rows)],
                            adsl_v.at[pl.ds(0, slice_rows)])
            adsl_v[pl.ds(slice_rows, 16)] = zeros_f

            def zero_chunk(i, carry):
                c = sid + i * _NS

                @pl.when(c < nzc)
                def _():
                    for acc in accs:
                        pltpu.sync_copy(zbuf, acc.at[pl.ds(c * 16, 16)])
                return carry

            lax.fori_loop(0, nzi, zero_chunk, 0)
            plsc.subcore_barrier()

            def compact_block(bbase, bsize, off):
                pltpu.sync_copy(dst_hbm.at[pl.ds(ebase + bbase, bsize)],
                                ebuf_d.at[pl.ds(0, bsize)])
                pltpu.sync_copy(src_hbm.at[pl.ds(ebase + bbase, bsize)],
                                ebuf_s.at[pl.ds(0, bsize)])

                def compact(ch, o):
                    d = ebuf_d[pl.ds(ch * 16, 16)]
                    s = ebuf_s[pl.ds(ch * 16, 16)]
                    m = (d >= lo) & (d < hi)
                    cs = plsc.cumsum(jnp.where(m, 1, 0))
                    pos = jnp.where(m, o + cs - 1, E_per + 15)
                    plsc.store_scatter(cdst_v, [pos], d - lo)
                    plsc.store_scatter(csrc_v, [pos], s)
                    return o + cs[15]

                return lax.fori_loop(0, bsize // 16, compact, off)

            def blk(b, off):
                return compact_block(b * _EBUF, _EBUF, off)

            off = lax.fori_loop(0, n_eb, blk, jnp.int32(0))
            if e_rem:
                off = compact_block(n_eb * _EBUF, e_rem, off)
            cdst_v[pl.ds(off, 16)] = jnp.full((16,), slice_rows, jnp.int32)
            csrc_v[pl.ds(off, 16)] = jnp.zeros((16,), jnp.int32)
            nch = (off + 15) // 16

            def proc(j, carry):
                sidx = csrc_v[pl.ds(j * 16, 16)]
                didx = cdst_v[pl.ds(j * 16, 16)]
                didxb[0, pl.ds(0, 16)] = didx
                pltpu.sync_copy(x_hbm.at[sidx], rowbuf)
                avals = plsc.load_gather(adsl_v, [didx])
                bvals = plsc.load_gather(bsrc_v, [sidx])
                attn = 1.0 / (1.0 + jnp.exp(-(avals + bvals)))
                for e in range(16):
                    ae = attn[e]
                    for fb in range(n_fb):
                        v = rowbuf[e, pl.ds(fb * 16, 16)]
                        msg = v * ae
                        p = jnp.exp(msg)
                        cp = fb * 16
                        cq = S + fb * 16
                        sbufs[cp // 128][e, pl.ds(cp % 128, 16)] = p
                        sbufs[cq // 128][e, pl.ds(cq % 128, 16)] = p * msg
                for k in range(n_acc):
                    pltpu.sync_copy(sbufs[k], accs[k].at[didxb.at[0]],
                                    add=True)
                return carry

            lax.fori_loop(0, nch, proc, 0)
            plsc.subcore_barrier()

            def writeback(i, carry):
                c = sid + i * _NS

                @pl.when(c < nzc)
                def _():
                    for k in range(n_acc):
                        pltpu.sync_copy(accs[k].at[pl.ds(c * 16, 16)],
                                        outs_hbm[k].at[pl.ds(lo + c * 16, 16)])
                return carry

            lax.fori_loop(0, nzi, writeback, 0)
            plsc.subcore_barrier()
            return 0

        lax.fori_loop(0, n_slices, slice_body, 0)

    f = pl.kernel(
        body,
        out_type=tuple(jax.ShapeDtypeStruct((2 * half_p, 128), jnp.float32)
                       for _ in range(n_acc)),
        mesh=mesh,
        compiler_params=pltpu.CompilerParams(needs_layout_passes=False),
        scratch_types=(
            [pltpu.VMEM((E_per + 16,), jnp.int32),
             pltpu.VMEM((E_per + 16,), jnp.int32),
             pltpu.VMEM((_EBUF,), jnp.int32),
             pltpu.VMEM((_EBUF,), jnp.int32),
             pltpu.VMEM((slice_rows + 16,), jnp.float32),
             pltpu.VMEM((n_src_pad,), jnp.float32),
             pltpu.VMEM((16, 128), jnp.float32),
             pltpu.VMEM((16, 128), jnp.float32),
             pltpu.VMEM((1, 16), jnp.int32)]
            + [pltpu.VMEM((16, 128), jnp.float32) for _ in range(n_acc)]
            + [pltpu.VMEM_SHARED((slice_rows + 16, 128), jnp.float32)
               for _ in range(n_acc)]
        ),
    )
    res = f(x128, adst, bsrc, src_e, dst_e)
    return list(res) if isinstance(res, (tuple, list)) else [res]


def _mish(x):
    return x * jnp.tanh(jax.nn.softplus(x))


def _row_spec(c):
    return pl.BlockSpec((_TILE, c), lambda i: (i, 0))


def _full_spec(a):
    return pl.BlockSpec(a.shape, lambda i: (0,) * a.ndim)


def _scalars_tc(x, cols):
    P, F = x.shape
    grid = (P // _TILE,)

    def body(x_ref, *refs):
        n = len(cols)
        wrefs = refs[:n]
        brefs = {i: r for i, r in zip(
            [i for i, c in enumerate(cols) if c[1] is not None],
            refs[n:n + sum(c[1] is not None for c in cols)])}
        orefs = refs[n + len(brefs):]
        xv = x_ref[...]
        for i in range(n):
            v = jnp.dot(xv, wrefs[i][...], preferred_element_type=jnp.float32)
            if i in brefs:
                v = v + brefs[i][...]
            orefs[i][...] = v

    args = [x] + [c[0] for c in cols] + [c[1].reshape(1, 1) for c in cols
                                         if c[1] is not None]
    in_specs = [_row_spec(F)] + [_full_spec(a) for a in args[1:]]
    outs = [jax.ShapeDtypeStruct((P, 1), jnp.float32) for _ in cols]
    out_specs = [_row_spec(1) for _ in cols]
    res = pl.pallas_call(body, grid=grid, out_shape=outs,
                         in_specs=in_specs, out_specs=out_specs)(*args)
    return [r.reshape(-1) for r in res]


def _aggr_from(acc_refs, S):
    if len(acc_refs) == 2:
        sump = acc_refs[0][...]
        sumpm = acc_refs[1][...]
    else:
        a = acc_refs[0][...]
        sump, sumpm = a[:, :S], a[:, S:]
    return sumpm / (sump + 1e-16)


def _dense_block_tc(accs, xdst, p, S, wsrc, wdst=None, be=None, pad128=False):
    n_acc = len(accs)
    P = accs[0].shape[0]
    T = xdst.shape[1]
    O = p["W2"].shape[0]
    grid = (P // _TILE,)
    W1a, W1b = p["W1"][:S], p["W1"][S:]
    b1 = p["b1"].reshape(1, O)
    b2 = p["b2"].reshape(1, O)
    have_dst = wdst is not None
    outs = [jax.ShapeDtypeStruct((P, O), jnp.float32),
            jax.ShapeDtypeStruct((P, 1), jnp.float32)]
    out_specs = [_row_spec(O), _row_spec(1)]
    if have_dst:
        outs.append(jax.ShapeDtypeStruct((P, 1), jnp.float32))
        out_specs.append(_row_spec(1))
    if pad128:
        outs.append(jax.ShapeDtypeStruct((P, 128), jnp.float32))
        out_specs.append(_row_spec(128))

    def body(*allrefs):
        acc_refs = allrefs[:n_acc]
        (x_ref, W1a_ref, W1b_ref, b1_ref, W2_ref, b2_ref,
         wsrc_ref) = allrefs[n_acc:n_acc + 7]
        refs = allrefs[n_acc + 7:]
        i = 0
        if have_dst:
            wdst_ref, be_ref = refs[0], refs[1]
            i = 2
        h_ref, bsrc_ref = refs[i], refs[i + 1]
        adst_ref = refs[i + 2] if have_dst else None
        pad_ref = refs[-1] if pad128 else None
        aggr = _aggr_from(acc_refs, S)
        h1 = _mish(jnp.dot(aggr, W1a_ref[...],
                           preferred_element_type=jnp.float32)
                   + jnp.dot(x_ref[...], W1b_ref[...],
                             preferred_element_type=jnp.float32)
                   + b1_ref[...])
        h = _mish(jnp.dot(h1, W2_ref[...],
                          preferred_element_type=jnp.float32) + b2_ref[...])
        h_ref[...] = h
        bsrc_ref[...] = jnp.dot(h, wsrc_ref[...],
                                preferred_element_type=jnp.float32)
        if have_dst:
            adst_ref[...] = jnp.dot(h, wdst_ref[...],
                                    preferred_element_type=jnp.float32) + be_ref[...]
        if pad128:
            pad_ref[...] = jnp.concatenate(
                [h, jnp.zeros((_TILE, 128 - O), jnp.float32)], axis=1)

    args = list(accs) + [xdst, W1a, W1b, b1, p["W2"], b2, wsrc]
    if have_dst:
        args += [wdst, be.reshape(1, 1)]
    in_specs = ([_row_spec(128)] * n_acc + [_row_spec(T)]
                + [_full_spec(a) for a in args[n_acc + 1:]])
    res = list(pl.pallas_call(body, grid=grid, out_shape=outs,
                              in_specs=in_specs, out_specs=out_specs)(*args))
    res[1] = res[1].reshape(-1)
    if have_dst:
        res[2] = res[2].reshape(-1)
    return res


def _final_tc(accs, hdst, h_of, h_ox, p, beta, coord, S):
    n_acc = len(accs)
    P = accs[0].shape[0]
    T = hdst.shape[1]
    O = p["W2"].shape[0]
    Inst = h_ox.shape[1]
    grid = (P // _TILE,)
    W1a, W1b = p["W1"][:S], p["W1"][S:]
    b1 = p["b1"].reshape(1, O)
    b2 = p["b2"].reshape(1, O)
    (Wb1, bb1), (Wb2, bb2), (Wb3, bb3) = beta
    (Wc1, bc1), (Wc2, bc2), (Wc3, bc3) = coord
    Wb1a, Wb1b = Wb1[:1], Wb1[1:]
    Wc1a, Wc1b = Wc1[:Inst], Wc1[Inst:]
    hidden = Wb2.shape[0]

    def body(*allrefs):
        acc_refs = allrefs[:n_acc]
        (x_ref, of_in_ref, ox_in_ref,
         W1a_ref, W1b_ref, b1_ref, W2_ref, b2_ref,
         Wb1a_ref, Wb1b_ref, bb1_ref, Wb2_ref, bb2_ref, Wb3_ref, bb3_ref,
         Wc1a_ref, Wc1b_ref, bc1_ref, Wc2_ref, bc2_ref, Wc3_ref, bc3_ref,
         h_ref, of_ref, ox_ref) = allrefs[n_acc:]
        aggr = _aggr_from(acc_refs, S)
        h1 = _mish(jnp.dot(aggr, W1a_ref[...],
                           preferred_element_type=jnp.float32)
                   + jnp.dot(x_ref[...], W1b_ref[...],
                             preferred_element_type=jnp.float32)
                   + b1_ref[...])
        h = _mish(jnp.dot(h1, W2_ref[...],
                          preferred_element_type=jnp.float32) + b2_ref[...])
        h_ref[...] = h
        u = _mish(jnp.dot(of_in_ref[...], Wb1a_ref[...],
                          preferred_element_type=jnp.float32)
                  + jnp.dot(h, Wb1b_ref[...],
                            preferred_element_type=jnp.float32)
                  + bb1_ref[...])
        u = _mish(jnp.dot(u, Wb2_ref[...],
                          preferred_element_type=jnp.float32) + bb2_ref[...])
        of_ref[...] = jax.nn.sigmoid(
            jnp.dot(u, Wb3_ref[...], preferred_element_type=jnp.float32)
            + bb3_ref[...])
        v = _mish(jnp.dot(ox_in_ref[...], Wc1a_ref[...],
                          preferred_element_type=jnp.float32)
                  + jnp.dot(h, Wc1b_ref[...],
                            preferred_element_type=jnp.float32)
                  + bc1_ref[...])
        v = _mish(jnp.dot(v, Wc2_ref[...],
                          preferred_element_type=jnp.float32) + bc2_ref[...])
        ox_ref[...] = jnp.dot(v, Wc3_ref[...],
                              preferred_element_type=jnp.float32) + bc3_ref[...]

    args = list(accs) + [hdst, h_of, h_ox, W1a, W1b, b1, p["W2"], b2,
                         Wb1a, Wb1b, bb1.reshape(1, hidden),
                         Wb2, bb2.reshape(1, hidden), Wb3, bb3.reshape(1, 1),
                         Wc1a, Wc1b, bc1.reshape(1, hidden),
                         Wc2, bc2.reshape(1, hidden), Wc3,
                         bc3.reshape(1, Inst)]
    in_specs = ([_row_spec(128)] * n_acc
                + [_row_spec(T), _row_spec(1), _row_spec(Inst)]
                + [_full_spec(a) for a in args[n_acc + 3:]])
    outs = [jax.ShapeDtypeStruct((P, O), jnp.float32),
            jax.ShapeDtypeStruct((P, 1), jnp.float32),
            jax.ShapeDtypeStruct((P, Inst), jnp.float32)]
    out_specs = [_row_spec(O), _row_spec(1), _row_spec(Inst)]
    return pl.pallas_call(body, grid=grid, out_shape=outs,
                          in_specs=in_specs, out_specs=out_specs)(*args)


def _pad_rows(x, P, C=None):
    C = C if C is not None else x.shape[1]
    return jnp.pad(x, ((0, P - x.shape[0]), (0, C - x.shape[1])))


def _pad_edges(src, dst):
    E = src.shape[0]
    E_pad = _round_up(E, 256)
    src_p = jnp.pad(src, (0, E_pad - E))
    dst_p = jnp.pad(dst, (0, E_pad - E), constant_values=-1)
    return src_p, dst_p


def kernel(h_x, sp_x, evt_x, h_of, h_ox, planar_edge_index, nexus_src,
           nexus_dst, sp_evt_src, sp_evt_dst, params):
    Nh, Hf = h_x.shape
    Nsp, Nf = sp_x.shape
    Ne, If_ = evt_x.shape
    half_h, half_sp, half_e = _half_rows(Nh), _half_rows(Nsp), _half_rows(Ne)
    P_h, P_sp, P_e = 2 * half_h, 2 * half_sp, 2 * half_e

    pp = params["plane"]
    pn = params["p2n"]
    pi = params["n2i"]
    pj = params["i2n"]
    pq = params["n2p"]

    h_x_p = _pad_rows(h_x, P_h)
    sp_x_p = _pad_rows(sp_x, P_sp)
    evt_x_p = _pad_rows(evt_x, P_e)

    src_pl, dst_pl = _pad_edges(planar_edge_index[0], planar_edge_index[1])
    src_nx, dst_nx = _pad_edges(nexus_src, nexus_dst)
    src_se, dst_se = _pad_edges(sp_evt_src, sp_evt_dst)
    src_es, dst_es = _pad_edges(sp_evt_dst, sp_evt_src)
    src_np, dst_np = _pad_edges(nexus_dst, nexus_src)

    bsrc_pl, adst_pl = _scalars_tc(
        h_x_p, [(pp["We"][Hf:], None), (pp["We"][:Hf], pp["be"])])
    (adst_p2n,) = _scalars_tc(sp_x_p, [(pn["We"][:Nf], pn["be"])])
    (adst_n2i,) = _scalars_tc(evt_x_p, [(pi["We"][:If_], pi["be"])])

    acc_pl = _sc_edge_phase(h_x_p, adst_pl, bsrc_pl, src_pl, dst_pl,
                            Hf, half_h, _num_slices(half_h, Hf))
    h_p, bsrc_p2n, adst_n2p = _dense_block_tc(
        acc_pl, h_x_p, pp, Hf, wsrc=pn["We"][Nf:],
        wdst=pq["We"][:Hf], be=pq["be"])

    acc_p2n = _sc_edge_phase(h_p, adst_p2n, bsrc_p2n, src_nx, dst_nx,
                             Hf, half_sp, _num_slices(half_sp, Hf))
    sp_p, bsrc_n2i, adst_i2n, sp128 = _dense_block_tc(
        acc_p2n, sp_x_p, pn, Hf, wsrc=pi["We"][If_:],
        wdst=pj["We"][:Nf], be=pj["be"], pad128=True)

    acc_n2i = _sc_edge_phase(sp128, adst_n2i, bsrc_n2i, src_se, dst_se,
                             Nf, half_e, _num_slices(half_e, Nf))
    evt_p, bsrc_i2n, evt128 = _dense_block_tc(
        acc_n2i, evt_x_p, pi, Nf, wsrc=pj["We"][Nf:], pad128=True)

    acc_i2n = _sc_edge_phase(evt128, adst_i2n, bsrc_i2n, src_es, dst_es,
                             If_, half_sp, _num_slices(half_sp, If_))
    sp2_p, bsrc_n2p, sp2128 = _dense_block_tc(
        acc_i2n, sp_p, pj, If_, wsrc=pq["We"][Hf:], pad128=True)

    acc_n2p = _sc_edge_phase(sp2128, adst_n2p, bsrc_n2p, src_np, dst_np,
                             Nf, half_h, _num_slices(half_h, Nf))
    h2_p, of_p, ox_p = _final_tc(acc_n2p, h_p, _pad_rows(h_of, P_h),
                                 _pad_rows(h_ox, P_h), pq,
                                 params["beta"], params["coord"], Nf)

    return (h2_p[:Nh], sp2_p[:Nsp], evt_p[:Ne], of_p[:Nh], ox_p[:Nh])


# speedup vs baseline: 4.1213x; 1.2527x over previous
import jax
import jax.numpy as jnp
from jax import lax
from jax.experimental import pallas as pl
from jax.experimental.pallas import tpu as pltpu
from jax.experimental.pallas import tpu_sc as plsc

_NS = 16
_TILE = 256
_ACC_BUDGET = {"plane": 3_300_000, "p2n": 5_300_000, "n2i": 1_400_000,
               "i2n": 2_700_000, "n2p": 3_300_000}
_EBUF = 1024


def _round_up(x, m):
    return -(-x // m) * m


def _half_rows(n):
    return _round_up(-(-n // 2), 2560)


def _num_slices(half_p, S, budget):
    k = 1
    while not (half_p % k == 0 and (half_p // k) % 16 == 0
               and (half_p // k + 16) * 2 * S * 4 <= budget):
        k += 1
    return k


def _sc_edge_phase(xt, adst, src_e, dst_e, S, half_p, n_slices):
    n_src_pad, SW = xt.shape
    E_pad = src_e.shape[0]
    E_per = E_pad // _NS
    slice_rows = half_p // n_slices
    assert slice_rows + 16 < (1 << 14) and n_src_pad < (1 << 16)
    nzc = slice_rows // 16
    nzi = -(-nzc // _NS)
    n_fb = S // 16
    C2 = 2 * S
    n_acc = C2 // 128
    n_eb = E_per // _EBUF
    e_rem = E_per - n_eb * _EBUF

    mesh = plsc.VectorSubcoreMesh(core_axis_name="c", subcore_axis_name="s",
                                  num_cores=2, num_subcores=_NS)

    def body(x_hbm, adst_hbm, src_hbm, dst_hbm, *rest):
        outs_hbm = rest[:n_acc]
        (cpk_v, ebuf_d, ebuf_s, adsl_v, zbuf, didxb) = rest[n_acc:n_acc + 6]
        k = n_acc + 6
        rowbufs = rest[k:k + 2]
        k += 2
        sbufs = [rest[k:k + n_acc], rest[k + n_acc:k + 2 * n_acc]]
        k += 2 * n_acc
        accs = rest[k:k + n_acc]
        k += n_acc
        gsems = rest[k:k + 2]
        k += 2
        ssems = [rest[k:k + n_acc], rest[k + n_acc:k + 2 * n_acc]]
        cid = lax.axis_index("c")
        sid = lax.axis_index("s")
        ebase = sid * E_per
        zeros_f = jnp.zeros((16,), jnp.float32)
        for r in range(16):
            for fb in range(8):
                zbuf[r, pl.ds(fb * 16, 16)] = zeros_f

        def slice_body(sl, _):
            lo = cid * half_p + sl * slice_rows
            hi = lo + slice_rows
            pltpu.sync_copy(adst_hbm.at[pl.ds(lo, slice_rows)],
                            adsl_v.at[pl.ds(0, slice_rows)])
            adsl_v[pl.ds(slice_rows, 16)] = zeros_f

            def zero_chunk(i, carry):
                c = sid + i * _NS

                @pl.when(c < nzc)
                def _():
                    for acc in accs:
                        pltpu.sync_copy(zbuf, acc.at[pl.ds(c * 16, 16)])
                return carry

            lax.fori_loop(0, nzi, zero_chunk, 0)
            plsc.subcore_barrier()

            def compact_block(bbase, bsize, off):
                pltpu.sync_copy(dst_hbm.at[pl.ds(ebase + bbase, bsize)],
                                ebuf_d.at[pl.ds(0, bsize)])
                pltpu.sync_copy(src_hbm.at[pl.ds(ebase + bbase, bsize)],
                                ebuf_s.at[pl.ds(0, bsize)])

                def compact(ch, o):
                    d = ebuf_d[pl.ds(ch * 16, 16)]
                    s = ebuf_s[pl.ds(ch * 16, 16)]
                    m = (d >= lo) & (d < hi)
                    cs = plsc.cumsum(jnp.where(m, 1, 0))
                    pos = jnp.where(m, o + cs - 1, E_per + 15)
                    pk = (s << 14) | (d - lo)
                    plsc.store_scatter(cpk_v, [pos], pk)
                    return o + cs[15]

                return lax.fori_loop(0, bsize // 16, compact, off)

            def blk(b, off):
                return compact_block(b * _EBUF, _EBUF, off)

            off = lax.fori_loop(0, n_eb, blk, jnp.int32(0))
            if e_rem:
                off = compact_block(n_eb * _EBUF, e_rem, off)
            cpk_v[pl.ds(off, 16)] = jnp.full((16,), slice_rows, jnp.int32)
            nch = (off + 15) // 16
            iota = lax.iota(jnp.int32, 16)
            col_b = jnp.full((16,), S, jnp.int32)

            def sidx_of(j):
                return lax.shift_right_logical(cpk_v[pl.ds(j * 16, 16)], 14)

            def issue_gather(j, par):
                pltpu.async_copy(x_hbm.at[sidx_of(j)], rowbufs[par],
                                 gsems[par])

            def wait_gather(j, par):
                pltpu.make_async_copy(x_hbm.at[sidx_of(j)], rowbufs[par],
                                      gsems[par]).wait()

            def wait_scatter(par):
                for a in range(n_acc):
                    pltpu.make_async_copy(
                        x_hbm.at[pl.ds(0, 16), pl.ds(0, 128)],
                        sbufs[par][a], ssems[par][a]).wait()

            def compute_issue(j, par):
                pk = cpk_v[pl.ds(j * 16, 16)]
                didx = pk & ((1 << 14) - 1)
                didxb[par, pl.ds(0, 16)] = didx
                avals = plsc.load_gather(adsl_v, [didx])
                bvals = plsc.load_gather(rowbufs[par], [iota, col_b])
                attn = 1.0 / (1.0 + jnp.exp(-(avals + bvals)))
                for e in range(16):
                    ae = attn[e]
                    for fb in range(n_fb):
                        v = rowbufs[par][e, pl.ds(fb * 16, 16)]
                        msg = v * ae
                        p = jnp.exp(msg)
                        cp = fb * 16
                        cq = S + fb * 16
                        sbufs[par][cp // 128][e, pl.ds(cp % 128, 16)] = p
                        sbufs[par][cq // 128][e, pl.ds(cq % 128, 16)] = p * msg
                for a in range(n_acc):
                    pltpu.async_copy(sbufs[par][a],
                                     accs[a].at[didxb.at[par]],
                                     ssems[par][a], add=True)

            @pl.when(nch > 0)
            def _():
                issue_gather(0, 0)

            def proc2(j2, carry):
                j0 = j2 * 2
                j1 = j0 + 1
                wait_gather(j0, 0)

                @pl.when(j1 < nch)
                def _():
                    issue_gather(j1, 1)

                @pl.when(j2 > 0)
                def _():
                    wait_scatter(0)

                compute_issue(j0, 0)

                @pl.when(j1 < nch)
                def _():
                    wait_gather(j1, 1)

                    @pl.when(j1 + 1 < nch)
                    def _():
                        issue_gather(j1 + 1, 0)

                    @pl.when(j2 > 0)
                    def _():
                        wait_scatter(1)

                    compute_issue(j1, 1)
                return carry

            lax.fori_loop(0, (nch + 1) // 2, proc2, 0)

            @pl.when(nch > 0)
            def _():
                wait_scatter(0)

            @pl.when(nch > 1)
            def _():
                wait_scatter(1)

            plsc.subcore_barrier()

            def writeback(i, carry):
                c = sid + i * _NS

                @pl.when(c < nzc)
                def _():
                    for k in range(n_acc):
                        pltpu.sync_copy(accs[k].at[pl.ds(c * 16, 16)],
                                        outs_hbm[k].at[pl.ds(lo + c * 16, 16)])
                return carry

            lax.fori_loop(0, nzi, writeback, 0)
            plsc.subcore_barrier()
            return 0

        lax.fori_loop(0, n_slices, slice_body, 0)

    f = pl.kernel(
        body,
        out_type=tuple(jax.ShapeDtypeStruct((2 * half_p, 128), jnp.float32)
                       for _ in range(n_acc)),
        mesh=mesh,
        compiler_params=pltpu.CompilerParams(needs_layout_passes=False),
        scratch_types=(
            [pltpu.VMEM((E_per + 16,), jnp.int32),
             pltpu.VMEM((_EBUF,), jnp.int32),
             pltpu.VMEM((_EBUF,), jnp.int32),
             pltpu.VMEM((slice_rows + 16,), jnp.float32),
             pltpu.VMEM((16, 128), jnp.float32),
             pltpu.VMEM((2, 16), jnp.int32)]
            + [pltpu.VMEM((16, SW), jnp.float32) for _ in range(2)]
            + [pltpu.VMEM((16, 128), jnp.float32)
               for _ in range(2 * n_acc)]
            + [pltpu.VMEM_SHARED((slice_rows + 16, 128), jnp.float32)
               for _ in range(n_acc)]
            + [pltpu.SemaphoreType.DMA for _ in range(2)]
            + [pltpu.SemaphoreType.DMA for _ in range(2 * n_acc)]
        ),
    )
    res = f(xt, adst, src_e, dst_e)
    return list(res) if isinstance(res, (tuple, list)) else [res]


def _mish(x):
    return x * jnp.tanh(jax.nn.softplus(x))


def _row_spec(c):
    return pl.BlockSpec((_TILE, c), lambda i: (i, 0))


def _full_spec(a):
    return pl.BlockSpec(a.shape, lambda i: (0,) * a.ndim)


def _table_tc(x, wsrc, sw):
    P, F = x.shape
    grid = (P // _TILE,)

    def body(x_ref, w_ref, o_ref):
        xv = x_ref[...]
        b = jnp.dot(xv, w_ref[...], preferred_element_type=jnp.float32)
        o_ref[...] = jnp.concatenate(
            [xv, b, jnp.zeros((_TILE, sw - F - 1), jnp.float32)], axis=1)

    return pl.pallas_call(
        body, grid=grid,
        out_shape=jax.ShapeDtypeStruct((P, sw), jnp.float32),
        in_specs=[_row_spec(F), _full_spec(wsrc)],
        out_specs=_row_spec(sw))(x, wsrc)


def _scalars_tc(x, cols):
    P, F = x.shape
    grid = (P // _TILE,)

    def body(x_ref, *refs):
        n = len(cols)
        wrefs = refs[:n]
        brefs = {i: r for i, r in zip(
            [i for i, c in enumerate(cols) if c[1] is not None],
            refs[n:n + sum(c[1] is not None for c in cols)])}
        orefs = refs[n + len(brefs):]
        xv = x_ref[...]
        for i in range(n):
            v = jnp.dot(xv, wrefs[i][...], preferred_element_type=jnp.float32)
            if i in brefs:
                v = v + brefs[i][...]
            orefs[i][...] = v

    args = [x] + [c[0] for c in cols] + [c[1].reshape(1, 1) for c in cols
                                         if c[1] is not None]
    in_specs = [_row_spec(F)] + [_full_spec(a) for a in args[1:]]
    outs = [jax.ShapeDtypeStruct((P, 1), jnp.float32) for _ in cols]
    out_specs = [_row_spec(1) for _ in cols]
    res = pl.pallas_call(body, grid=grid, out_shape=outs,
                         in_specs=in_specs, out_specs=out_specs)(*args)
    return [r.reshape(-1) for r in res]


def _aggr_from(acc_refs, S):
    if len(acc_refs) == 2:
        sump = acc_refs[0][...]
        sumpm = acc_refs[1][...]
    else:
        a = acc_refs[0][...]
        sump, sumpm = a[:, :S], a[:, S:]
    return sumpm / (sump + 1e-16)


def _dense_block_tc(accs, xdst, p, S, wsrc, sw_next, wdst=None, be=None):
    n_acc = len(accs)
    P = accs[0].shape[0]
    T = xdst.shape[1]
    O = p["W2"].shape[0]
    grid = (P // _TILE,)
    W1a, W1b = p["W1"][:S], p["W1"][S:]
    b1 = p["b1"].reshape(1, O)
    b2 = p["b2"].reshape(1, O)
    have_dst = wdst is not None
    outs = [jax.ShapeDtypeStruct((P, O), jnp.float32),
            jax.ShapeDtypeStruct((P, sw_next), jnp.float32)]
    out_specs = [_row_spec(O), _row_spec(sw_next)]
    if have_dst:
        outs.append(jax.ShapeDtypeStruct((P, 1), jnp.float32))
        out_specs.append(_row_spec(1))

    def body(*allrefs):
        acc_refs = allrefs[:n_acc]
        (x_ref, W1a_ref, W1b_ref, b1_ref, W2_ref, b2_ref,
         wsrc_ref) = allrefs[n_acc:n_acc + 7]
        refs = allrefs[n_acc + 7:]
        i = 0
        if have_dst:
            wdst_ref, be_ref = refs[0], refs[1]
            i = 2
        h_ref, xt_ref = refs[i], refs[i + 1]
        adst_ref = refs[i + 2] if have_dst else None
        aggr = _aggr_from(acc_refs, S)
        h1 = _mish(jnp.dot(aggr, W1a_ref[...],
                           preferred_element_type=jnp.float32)
                   + jnp.dot(x_ref[...], W1b_ref[...],
                             preferred_element_type=jnp.float32)
                   + b1_ref[...])
        h = _mish(jnp.dot(h1, W2_ref[...],
                          preferred_element_type=jnp.float32) + b2_ref[...])
        h_ref[...] = h
        bsrc = jnp.dot(h, wsrc_ref[...], preferred_element_type=jnp.float32)
        xt_ref[...] = jnp.concatenate(
            [h, bsrc, jnp.zeros((_TILE, sw_next - O - 1), jnp.float32)],
            axis=1)
        if have_dst:
            adst_ref[...] = jnp.dot(h, wdst_ref[...],
                                    preferred_element_type=jnp.float32) + be_ref[...]

    args = list(accs) + [xdst, W1a, W1b, b1, p["W2"], b2, wsrc]
    if have_dst:
        args += [wdst, be.reshape(1, 1)]
    in_specs = ([_row_spec(128)] * n_acc + [_row_spec(T)]
                + [_full_spec(a) for a in args[n_acc + 1:]])
    res = list(pl.pallas_call(body, grid=grid, out_shape=outs,
                              in_specs=in_specs, out_specs=out_specs)(*args))
    if have_dst:
        res[2] = res[2].reshape(-1)
    return res


def _final_tc(accs, hdst, h_of, h_ox, p, beta, coord, S):
    n_acc = len(accs)
    P = accs[0].shape[0]
    T = hdst.shape[1]
    O = p["W2"].shape[0]
    Inst = h_ox.shape[1]
    grid = (P // _TILE,)
    W1a, W1b = p["W1"][:S], p["W1"][S:]
    b1 = p["b1"].reshape(1, O)
    b2 = p["b2"].reshape(1, O)
    (Wb1, bb1), (Wb2, bb2), (Wb3, bb3) = beta
    (Wc1, bc1), (Wc2, bc2), (Wc3, bc3) = coord
    Wb1a, Wb1b = Wb1[:1], Wb1[1:]
    Wc1a, Wc1b = Wc1[:Inst], Wc1[Inst:]
    hidden = Wb2.shape[0]

    def body(*allrefs):
        acc_refs = allrefs[:n_acc]
        (x_ref, of_in_ref, ox_in_ref,
         W1a_ref, W1b_ref, b1_ref, W2_ref, b2_ref,
         Wb1a_ref, Wb1b_ref, bb1_ref, Wb2_ref, bb2_ref, Wb3_ref, bb3_ref,
         Wc1a_ref, Wc1b_ref, bc1_ref, Wc2_ref, bc2_ref, Wc3_ref, bc3_ref,
         h_ref, of_ref, ox_ref) = allrefs[n_acc:]
        aggr = _aggr_from(acc_refs, S)
        h1 = _mish(jnp.dot(aggr, W1a_ref[...],
                           preferred_element_type=jnp.float32)
                   + jnp.dot(x_ref[...], W1b_ref[...],
                             preferred_element_type=jnp.float32)
                   + b1_ref[...])
        h = _mish(jnp.dot(h1, W2_ref[...],
                          preferred_element_type=jnp.float32) + b2_ref[...])
        h_ref[...] = h
        u = _mish(jnp.dot(of_in_ref[...], Wb1a_ref[...],
                          preferred_element_type=jnp.float32)
                  + jnp.dot(h, Wb1b_ref[...],
                            preferred_element_type=jnp.float32)
                  + bb1_ref[...])
        u = _mish(jnp.dot(u, Wb2_ref[...],
                          preferred_element_type=jnp.float32) + bb2_ref[...])
        of_ref[...] = jax.nn.sigmoid(
            jnp.dot(u, Wb3_ref[...], preferred_element_type=jnp.float32)
            + bb3_ref[...])
        v = _mish(jnp.dot(ox_in_ref[...], Wc1a_ref[...],
                          preferred_element_type=jnp.float32)
                  + jnp.dot(h, Wc1b_ref[...],
                            preferred_element_type=jnp.float32)
                  + bc1_ref[...])
        v = _mish(jnp.dot(v, Wc2_ref[...],
                          preferred_element_type=jnp.float32) + bc2_ref[...])
        ox_ref[...] = jnp.dot(v, Wc3_ref[...],
                              preferred_element_type=jnp.float32) + bc3_ref[...]

    args = list(accs) + [hdst, h_of, h_ox, W1a, W1b, b1, p["W2"], b2,
                         Wb1a, Wb1b, bb1.reshape(1, hidden),
                         Wb2, bb2.reshape(1, hidden), Wb3, bb3.reshape(1, 1),
                         Wc1a, Wc1b, bc1.reshape(1, hidden),
                         Wc2, bc2.reshape(1, hidden), Wc3,
                         bc3.reshape(1, Inst)]
    in_specs = ([_row_spec(128)] * n_acc
                + [_row_spec(T), _row_spec(1), _row_spec(Inst)]
                + [_full_spec(a) for a in args[n_acc + 3:]])
    outs = [jax.ShapeDtypeStruct((P, O), jnp.float32),
            jax.ShapeDtypeStruct((P, 1), jnp.float32),
            jax.ShapeDtypeStruct((P, Inst), jnp.float32)]
    out_specs = [_row_spec(O), _row_spec(1), _row_spec(Inst)]
    return pl.pallas_call(body, grid=grid, out_shape=outs,
                          in_specs=in_specs, out_specs=out_specs)(*args)


def _pad_rows(x, P, C=None):
    C = C if C is not None else x.shape[1]
    return jnp.pad(x, ((0, P - x.shape[0]), (0, C - x.shape[1])))


def _pad_edges(src, dst):
    E = src.shape[0]
    E_pad = _round_up(E, 256)
    src_p = jnp.pad(src, (0, E_pad - E))
    dst_p = jnp.pad(dst, (0, E_pad - E), constant_values=-1)
    return src_p, dst_p


def kernel(h_x, sp_x, evt_x, h_of, h_ox, planar_edge_index, nexus_src,
           nexus_dst, sp_evt_src, sp_evt_dst, params):
    Nh, Hf = h_x.shape
    Nsp, Nf = sp_x.shape
    Ne, If_ = evt_x.shape
    half_h, half_sp, half_e = _half_rows(Nh), _half_rows(Nsp), _half_rows(Ne)
    P_h, P_sp, P_e = 2 * half_h, 2 * half_sp, 2 * half_e

    pp = params["plane"]
    pn = params["p2n"]
    pi = params["n2i"]
    pj = params["i2n"]
    pq = params["n2p"]

    h_x_p = _pad_rows(h_x, P_h)
    sp_x_p = _pad_rows(sp_x, P_sp)
    evt_x_p = _pad_rows(evt_x, P_e)

    src_pl, dst_pl = _pad_edges(planar_edge_index[0], planar_edge_index[1])
    src_nx, dst_nx = _pad_edges(nexus_src, nexus_dst)
    src_se, dst_se = _pad_edges(sp_evt_src, sp_evt_dst)
    src_es, dst_es = _pad_edges(sp_evt_dst, sp_evt_src)
    src_np, dst_np = _pad_edges(nexus_dst, nexus_src)

    xt_pl = _table_tc(h_x_p, pp["We"][Hf:], 256)
    (adst_pl,) = _scalars_tc(h_x_p, [(pp["We"][:Hf], pp["be"])])
    (adst_p2n,) = _scalars_tc(sp_x_p, [(pn["We"][:Nf], pn["be"])])
    (adst_n2i,) = _scalars_tc(evt_x_p, [(pi["We"][:If_], pi["be"])])

    acc_pl = _sc_edge_phase(xt_pl, adst_pl, src_pl, dst_pl, Hf, half_h,
                            _num_slices(half_h, Hf, _ACC_BUDGET["plane"]))
    h_p, xt_p2n, adst_n2p = _dense_block_tc(
        acc_pl, h_x_p, pp, Hf, wsrc=pn["We"][Nf:], sw_next=256,
        wdst=pq["We"][:Hf], be=pq["be"])

    acc_p2n = _sc_edge_phase(xt_p2n, adst_p2n, src_nx, dst_nx, Hf, half_sp,
                             _num_slices(half_sp, Hf, _ACC_BUDGET["p2n"]))
    sp_p, xt_n2i, adst_i2n = _dense_block_tc(
        acc_p2n, sp_x_p, pn, Hf, wsrc=pi["We"][If_:], sw_next=128,
        wdst=pj["We"][:Nf], be=pj["be"])

    acc_n2i = _sc_edge_phase(xt_n2i, adst_n2i, src_se, dst_se, Nf, half_e,
                             _num_slices(half_e, Nf, _ACC_BUDGET["n2i"]))
    evt_p, xt_i2n = _dense_block_tc(
        acc_n2i, evt_x_p, pi, Nf, wsrc=pj["We"][Nf:], sw_next=128)

    acc_i2n = _sc_edge_phase(xt_i2n, adst_i2n, src_es, dst_es, If_, half_sp,
                             _num_slices(half_sp, If_, _ACC_BUDGET["i2n"]))
    sp2_p, xt_n2p = _dense_block_tc(
        acc_i2n, sp_p, pj, If_, wsrc=pq["We"][Hf:], sw_next=128)

    acc_n2p = _sc_edge_phase(xt_n2p, adst_n2p, src_np, dst_np, Nf, half_h,
                             _num_slices(half_h, Nf, _ACC_BUDGET["n2p"]))
    h2_p, of_p, ox_p = _final_tc(acc_n2p, h_p, _pad_rows(h_of, P_h),
                                 _pad_rows(h_ox, P_h), pq,
                                 params["beta"], params["coord"], Nf)

    return (h2_p[:Nh], sp2_p[:Nsp], evt_p[:Ne], of_p[:Nh], ox_p[:Nh])


# speedup vs baseline: 4.3802x; 1.0628x over previous
import jax
import jax.numpy as jnp
from jax import lax
from jax.experimental import pallas as pl
from jax.experimental.pallas import tpu as pltpu
from jax.experimental.pallas import tpu_sc as plsc

_NS = 16
_TILE = 256
_ACC_BUDGET = {"plane": 3_300_000, "p2n": 5_300_000, "n2i": 1_400_000,
               "i2n": 2_700_000, "n2p": 3_300_000}
_EBUF = 1024


def _round_up(x, m):
    return -(-x // m) * m


def _half_rows(n):
    return _round_up(-(-n // 2), 2560)


def _num_slices(half_p, S, budget):
    k = 1
    while not (half_p % k == 0 and (half_p // k) % 16 == 0
               and (half_p // k + 16) * 2 * S * 4 <= budget):
        k += 1
    return k


def _sc_edge_phase(xt, adst, src_e, dst_e, S, half_p, n_slices):
    n_src_pad, SW = xt.shape
    E_pad = src_e.shape[0]
    E_per = E_pad // _NS
    slice_rows = half_p // n_slices
    assert slice_rows + 16 < (1 << 14) and n_src_pad < (1 << 16)
    zrows = 64 if slice_rows % 64 == 0 else 16
    nzc = slice_rows // zrows
    nzi = -(-nzc // _NS)
    n_fb = S // 16
    C2 = 2 * S
    n_acc = C2 // 128
    n_eb = E_per // _EBUF
    e_rem = E_per - n_eb * _EBUF

    mesh = plsc.VectorSubcoreMesh(core_axis_name="c", subcore_axis_name="s",
                                  num_cores=2, num_subcores=_NS)

    def body(x_hbm, adst_hbm, src_hbm, dst_hbm, *rest):
        outs_hbm = rest[:n_acc]
        (cpk_v, ebuf_d, ebuf_s, adsl_v, zbuf, didxb) = rest[n_acc:n_acc + 6]
        k = n_acc + 6
        rowbufs = rest[k:k + 2]
        k += 2
        sbufs = [rest[k:k + n_acc], rest[k + n_acc:k + 2 * n_acc]]
        k += 2 * n_acc
        accs = rest[k:k + n_acc]
        k += n_acc
        gsems = rest[k:k + 2]
        k += 2
        ssems = [rest[k:k + n_acc], rest[k + n_acc:k + 2 * n_acc]]
        cid = lax.axis_index("c")
        sid = lax.axis_index("s")
        ebase = sid * E_per
        zeros_f = jnp.zeros((16,), jnp.float32)
        for r in range(zrows):
            for fb in range(8):
                zbuf[r, pl.ds(fb * 16, 16)] = zeros_f

        def slice_body(sl, _):
            lo = cid * half_p + sl * slice_rows
            hi = lo + slice_rows
            pltpu.sync_copy(adst_hbm.at[pl.ds(lo, slice_rows)],
                            adsl_v.at[pl.ds(0, slice_rows)])
            adsl_v[pl.ds(slice_rows, 16)] = zeros_f

            def zero_chunk(i, carry):
                c = sid + i * _NS

                @pl.when(c < nzc)
                def _():
                    for acc in accs:
                        pltpu.sync_copy(zbuf, acc.at[pl.ds(c * zrows, zrows)])
                return carry

            lax.fori_loop(0, nzi, zero_chunk, 0)
            plsc.subcore_barrier()

            def compact_block(bbase, bsize, off):
                pltpu.sync_copy(dst_hbm.at[pl.ds(ebase + bbase, bsize)],
                                ebuf_d.at[pl.ds(0, bsize)])
                pltpu.sync_copy(src_hbm.at[pl.ds(ebase + bbase, bsize)],
                                ebuf_s.at[pl.ds(0, bsize)])

                def compact(ch, o):
                    d = ebuf_d[pl.ds(ch * 16, 16)]
                    s = ebuf_s[pl.ds(ch * 16, 16)]
                    m = (d >= lo) & (d < hi)
                    cs = plsc.cumsum(jnp.where(m, 1, 0))
                    pos = jnp.where(m, o + cs - 1, E_per + 15)
                    pk = (s << 14) | (d - lo)
                    plsc.store_scatter(cpk_v, [pos], pk)
                    return o + cs[15]

                return lax.fori_loop(0, bsize // 16, compact, off)

            def blk(b, off):
                return compact_block(b * _EBUF, _EBUF, off)

            off = lax.fori_loop(0, n_eb, blk, jnp.int32(0))
            if e_rem:
                off = compact_block(n_eb * _EBUF, e_rem, off)
            cpk_v[pl.ds(off, 16)] = jnp.full((16,), slice_rows, jnp.int32)
            nch = (off + 15) // 16
            iota = lax.iota(jnp.int32, 16)
            col_b = jnp.full((16,), S, jnp.int32)

            def sidx_of(j):
                return lax.shift_right_logical(cpk_v[pl.ds(j * 16, 16)], 14)

            def issue_gather(j, par):
                pltpu.async_copy(x_hbm.at[sidx_of(j)], rowbufs[par],
                                 gsems[par])

            def wait_gather(j, par):
                pltpu.make_async_copy(x_hbm.at[sidx_of(j)], rowbufs[par],
                                      gsems[par]).wait()

            def wait_scatter(par):
                for a in range(n_acc):
                    pltpu.make_async_copy(
                        x_hbm.at[pl.ds(0, 16), pl.ds(0, 128)],
                        sbufs[par][a], ssems[par][a]).wait()

            def compute_issue(j, par):
                pk = cpk_v[pl.ds(j * 16, 16)]
                didx = pk & ((1 << 14) - 1)
                didxb[par, pl.ds(0, 16)] = didx
                avals = plsc.load_gather(adsl_v, [didx])
                bvals = plsc.load_gather(rowbufs[par], [iota, col_b])
                attn = 1.0 / (1.0 + jnp.exp(-(avals + bvals)))
                for e in range(16):
                    ae = attn[e]
                    for fb in range(n_fb):
                        v = rowbufs[par][e, pl.ds(fb * 16, 16)]
                        msg = v * ae
                        p = jnp.exp(msg)
                        cp = fb * 16
                        cq = S + fb * 16
                        sbufs[par][cp // 128][e, pl.ds(cp % 128, 16)] = p
                        sbufs[par][cq // 128][e, pl.ds(cq % 128, 16)] = p * msg
                for a in range(n_acc):
                    pltpu.async_copy(sbufs[par][a],
                                     accs[a].at[didxb.at[par]],
                                     ssems[par][a], add=True)

            @pl.when(nch > 0)
            def _():
                issue_gather(0, 0)

            def proc2(j2, carry):
                j0 = j2 * 2
                j1 = j0 + 1
                wait_gather(j0, 0)

                @pl.when(j1 < nch)
                def _():
                    issue_gather(j1, 1)

                @pl.when(j2 > 0)
                def _():
                    wait_scatter(0)

                compute_issue(j0, 0)

                @pl.when(j1 < nch)
                def _():
                    wait_gather(j1, 1)

                    @pl.when(j1 + 1 < nch)
                    def _():
                        issue_gather(j1 + 1, 0)

                    @pl.when(j2 > 0)
                    def _():
                        wait_scatter(1)

                    compute_issue(j1, 1)
                return carry

            lax.fori_loop(0, (nch + 1) // 2, proc2, 0)

            @pl.when(nch > 0)
            def _():
                wait_scatter(0)

            @pl.when(nch > 1)
            def _():
                wait_scatter(1)

            plsc.subcore_barrier()

            def writeback(i, carry):
                c = sid + i * _NS

                @pl.when(c < nzc)
                def _():
                    for k in range(n_acc):
                        pltpu.sync_copy(
                            accs[k].at[pl.ds(c * zrows, zrows)],
                            outs_hbm[k].at[pl.ds(lo + c * zrows, zrows)])
                return carry

            lax.fori_loop(0, nzi, writeback, 0)
            plsc.subcore_barrier()
            return 0

        lax.fori_loop(0, n_slices, slice_body, 0)

    f = pl.kernel(
        body,
        out_type=tuple(jax.ShapeDtypeStruct((2 * half_p, 128), jnp.float32)
                       for _ in range(n_acc)),
        mesh=mesh,
        compiler_params=pltpu.CompilerParams(needs_layout_passes=False),
        scratch_types=(
            [pltpu.VMEM((E_per + 16,), jnp.int32),
             pltpu.VMEM((_EBUF,), jnp.int32),
             pltpu.VMEM((_EBUF,), jnp.int32),
             pltpu.VMEM((slice_rows + 16,), jnp.float32),
             pltpu.VMEM((zrows, 128), jnp.float32),
             pltpu.VMEM((2, 16), jnp.int32)]
            + [pltpu.VMEM((16, SW), jnp.float32) for _ in range(2)]
            + [pltpu.VMEM((16, 128), jnp.float32)
               for _ in range(2 * n_acc)]
            + [pltpu.VMEM_SHARED((slice_rows + 16, 128), jnp.float32)
               for _ in range(n_acc)]
            + [pltpu.SemaphoreType.DMA for _ in range(2)]
            + [pltpu.SemaphoreType.DMA for _ in range(2 * n_acc)]
        ),
    )
    res = f(xt, adst, src_e, dst_e)
    return list(res) if isinstance(res, (tuple, list)) else [res]


def _mish(x):
    return x * jnp.tanh(jax.nn.softplus(x))


def _row_spec(c):
    return pl.BlockSpec((_TILE, c), lambda i: (i, 0))


def _full_spec(a):
    return pl.BlockSpec(a.shape, lambda i: (0,) * a.ndim)


def _table_tc(x, wsrc, sw):
    P, F = x.shape
    grid = (P // _TILE,)

    def body(x_ref, w_ref, o_ref):
        xv = x_ref[...]
        b = jnp.dot(xv, w_ref[...], preferred_element_type=jnp.float32)
        o_ref[...] = jnp.concatenate(
            [xv, b, jnp.zeros((_TILE, sw - F - 1), jnp.float32)], axis=1)

    return pl.pallas_call(
        body, grid=grid,
        out_shape=jax.ShapeDtypeStruct((P, sw), jnp.float32),
        in_specs=[_row_spec(F), _full_spec(wsrc)],
        out_specs=_row_spec(sw))(x, wsrc)


def _scalars_tc(x, cols):
    P, F = x.shape
    grid = (P // _TILE,)

    def body(x_ref, *refs):
        n = len(cols)
        wrefs = refs[:n]
        brefs = {i: r for i, r in zip(
            [i for i, c in enumerate(cols) if c[1] is not None],
            refs[n:n + sum(c[1] is not None for c in cols)])}
        orefs = refs[n + len(brefs):]
        xv = x_ref[...]
        for i in range(n):
            v = jnp.dot(xv, wrefs[i][...], preferred_element_type=jnp.float32)
            if i in brefs:
                v = v + brefs[i][...]
            orefs[i][...] = v

    args = [x] + [c[0] for c in cols] + [c[1].reshape(1, 1) for c in cols
                                         if c[1] is not None]
    in_specs = [_row_spec(F)] + [_full_spec(a) for a in args[1:]]
    outs = [jax.ShapeDtypeStruct((P, 1), jnp.float32) for _ in cols]
    out_specs = [_row_spec(1) for _ in cols]
    res = pl.pallas_call(body, grid=grid, out_shape=outs,
                         in_specs=in_specs, out_specs=out_specs)(*args)
    return [r.reshape(-1) for r in res]


def _aggr_from(acc_refs, S):
    if len(acc_refs) == 2:
        sump = acc_refs[0][...]
        sumpm = acc_refs[1][...]
    else:
        a = acc_refs[0][...]
        sump, sumpm = a[:, :S], a[:, S:]
    return sumpm / (sump + 1e-16)


def _dense_block_tc(accs, xdst, p, S, wsrc, sw_next, wdst=None, be=None):
    n_acc = len(accs)
    P = accs[0].shape[0]
    T = xdst.shape[1]
    O = p["W2"].shape[0]
    grid = (P // _TILE,)
    W1a, W1b = p["W1"][:S], p["W1"][S:]
    b1 = p["b1"].reshape(1, O)
    b2 = p["b2"].reshape(1, O)
    have_dst = wdst is not None
    outs = [jax.ShapeDtypeStruct((P, O), jnp.float32),
            jax.ShapeDtypeStruct((P, sw_next), jnp.float32)]
    out_specs = [_row_spec(O), _row_spec(sw_next)]
    if have_dst:
        outs.append(jax.ShapeDtypeStruct((P, 1), jnp.float32))
        out_specs.append(_row_spec(1))

    def body(*allrefs):
        acc_refs = allrefs[:n_acc]
        (x_ref, W1a_ref, W1b_ref, b1_ref, W2_ref, b2_ref,
         wsrc_ref) = allrefs[n_acc:n_acc + 7]
        refs = allrefs[n_acc + 7:]
        i = 0
        if have_dst:
            wdst_ref, be_ref = refs[0], refs[1]
            i = 2
        h_ref, xt_ref = refs[i], refs[i + 1]
        adst_ref = refs[i + 2] if have_dst else None
        aggr = _aggr_from(acc_refs, S)
        h1 = _mish(jnp.dot(aggr, W1a_ref[...],
                           preferred_element_type=jnp.float32)
                   + jnp.dot(x_ref[...], W1b_ref[...],
                             preferred_element_type=jnp.float32)
                   + b1_ref[...])
        h = _mish(jnp.dot(h1, W2_ref[...],
                          preferred_element_type=jnp.float32) + b2_ref[...])
        h_ref[...] = h
        bsrc = jnp.dot(h, wsrc_ref[...], preferred_element_type=jnp.float32)
        xt_ref[...] = jnp.concatenate(
            [h, bsrc, jnp.zeros((_TILE, sw_next - O - 1), jnp.float32)],
            axis=1)
        if have_dst:
            adst_ref[...] = jnp.dot(h, wdst_ref[...],
                                    preferred_element_type=jnp.float32) + be_ref[...]

    args = list(accs) + [xdst, W1a, W1b, b1, p["W2"], b2, wsrc]
    if have_dst:
        args += [wdst, be.reshape(1, 1)]
    in_specs = ([_row_spec(128)] * n_acc + [_row_spec(T)]
                + [_full_spec(a) for a in args[n_acc + 1:]])
    res = list(pl.pallas_call(body, grid=grid, out_shape=outs,
                              in_specs=in_specs, out_specs=out_specs)(*args))
    if have_dst:
        res[2] = res[2].reshape(-1)
    return res


def _final_tc(accs, hdst, h_of, h_ox, p, beta, coord, S):
    n_acc = len(accs)
    P = accs[0].shape[0]
    T = hdst.shape[1]
    O = p["W2"].shape[0]
    Inst = h_ox.shape[1]
    grid = (P // _TILE,)
    W1a, W1b = p["W1"][:S], p["W1"][S:]
    b1 = p["b1"].reshape(1, O)
    b2 = p["b2"].reshape(1, O)
    (Wb1, bb1), (Wb2, bb2), (Wb3, bb3) = beta
    (Wc1, bc1), (Wc2, bc2), (Wc3, bc3) = coord
    Wb1a, Wb1b = Wb1[:1], Wb1[1:]
    Wc1a, Wc1b = Wc1[:Inst], Wc1[Inst:]
    hidden = Wb2.shape[0]

    def body(*allrefs):
        acc_refs = allrefs[:n_acc]
        (x_ref, of_in_ref, ox_in_ref,
         W1a_ref, W1b_ref, b1_ref, W2_ref, b2_ref,
         Wb1a_ref, Wb1b_ref, bb1_ref, Wb2_ref, bb2_ref, Wb3_ref, bb3_ref,
         Wc1a_ref, Wc1b_ref, bc1_ref, Wc2_ref, bc2_ref, Wc3_ref, bc3_ref,
         h_ref, of_ref, ox_ref) = allrefs[n_acc:]
        aggr = _aggr_from(acc_refs, S)
        h1 = _mish(jnp.dot(aggr, W1a_ref[...],
                           preferred_element_type=jnp.float32)
                   + jnp.dot(x_ref[...], W1b_ref[...],
                             preferred_element_type=jnp.float32)
                   + b1_ref[...])
        h = _mish(jnp.dot(h1, W2_ref[...],
                          preferred_element_type=jnp.float32) + b2_ref[...])
        h_ref[...] = h
        u = _mish(jnp.dot(of_in_ref[...], Wb1a_ref[...],
                          preferred_element_type=jnp.float32)
                  + jnp.dot(h, Wb1b_ref[...],
                            preferred_element_type=jnp.float32)
                  + bb1_ref[...])
        u = _mish(jnp.dot(u, Wb2_ref[...],
                          preferred_element_type=jnp.float32) + bb2_ref[...])
        of_ref[...] = jax.nn.sigmoid(
            jnp.dot(u, Wb3_ref[...], preferred_element_type=jnp.float32)
            + bb3_ref[...])
        v = _mish(jnp.dot(ox_in_ref[...], Wc1a_ref[...],
                          preferred_element_type=jnp.float32)
                  + jnp.dot(h, Wc1b_ref[...],
                            preferred_element_type=jnp.float32)
                  + bc1_ref[...])
        v = _mish(jnp.dot(v, Wc2_ref[...],
                          preferred_element_type=jnp.float32) + bc2_ref[...])
        ox_ref[...] = jnp.dot(v, Wc3_ref[...],
                              preferred_element_type=jnp.float32) + bc3_ref[...]

    args = list(accs) + [hdst, h_of, h_ox, W1a, W1b, b1, p["W2"], b2,
                         Wb1a, Wb1b, bb1.reshape(1, hidden),
                         Wb2, bb2.reshape(1, hidden), Wb3, bb3.reshape(1, 1),
                         Wc1a, Wc1b, bc1.reshape(1, hidden),
                         Wc2, bc2.reshape(1, hidden), Wc3,
                         bc3.reshape(1, Inst)]
    in_specs = ([_row_spec(128)] * n_acc
                + [_row_spec(T), _row_spec(1), _row_spec(Inst)]
                + [_full_spec(a) for a in args[n_acc + 3:]])
    outs = [jax.ShapeDtypeStruct((P, O), jnp.float32),
            jax.ShapeDtypeStruct((P, 1), jnp.float32),
            jax.ShapeDtypeStruct((P, Inst), jnp.float32)]
    out_specs = [_row_spec(O), _row_spec(1), _row_spec(Inst)]
    return pl.pallas_call(body, grid=grid, out_shape=outs,
                          in_specs=in_specs, out_specs=out_specs)(*args)


def _pad_rows(x, P, C=None):
    C = C if C is not None else x.shape[1]
    return jnp.pad(x, ((0, P - x.shape[0]), (0, C - x.shape[1])))


def _pad_edges(src, dst):
    E = src.shape[0]
    E_pad = _round_up(E, 256)
    src_p = jnp.pad(src, (0, E_pad - E))
    dst_p = jnp.pad(dst, (0, E_pad - E), constant_values=-1)
    return src_p, dst_p


def kernel(h_x, sp_x, evt_x, h_of, h_ox, planar_edge_index, nexus_src,
           nexus_dst, sp_evt_src, sp_evt_dst, params):
    Nh, Hf = h_x.shape
    Nsp, Nf = sp_x.shape
    Ne, If_ = evt_x.shape
    half_h, half_sp, half_e = _half_rows(Nh), _half_rows(Nsp), _half_rows(Ne)
    P_h, P_sp, P_e = 2 * half_h, 2 * half_sp, 2 * half_e

    pp = params["plane"]
    pn = params["p2n"]
    pi = params["n2i"]
    pj = params["i2n"]
    pq = params["n2p"]

    h_x_p = _pad_rows(h_x, P_h)
    sp_x_p = _pad_rows(sp_x, P_sp)
    evt_x_p = _pad_rows(evt_x, P_e)

    src_pl, dst_pl = _pad_edges(planar_edge_index[0], planar_edge_index[1])
    src_nx, dst_nx = _pad_edges(nexus_src, nexus_dst)
    src_se, dst_se = _pad_edges(sp_evt_src, sp_evt_dst)
    src_es, dst_es = _pad_edges(sp_evt_dst, sp_evt_src)
    src_np, dst_np = _pad_edges(nexus_dst, nexus_src)

    xt_pl = _table_tc(h_x_p, pp["We"][Hf:], 256)
    (adst_pl,) = _scalars_tc(h_x_p, [(pp["We"][:Hf], pp["be"])])
    (adst_p2n,) = _scalars_tc(sp_x_p, [(pn["We"][:Nf], pn["be"])])
    (adst_n2i,) = _scalars_tc(evt_x_p, [(pi["We"][:If_], pi["be"])])

    acc_pl = _sc_edge_phase(xt_pl, adst_pl, src_pl, dst_pl, Hf, half_h,
                            _num_slices(half_h, Hf, _ACC_BUDGET["plane"]))
    h_p, xt_p2n, adst_n2p = _dense_block_tc(
        acc_pl, h_x_p, pp, Hf, wsrc=pn["We"][Nf:], sw_next=256,
        wdst=pq["We"][:Hf], be=pq["be"])

    acc_p2n = _sc_edge_phase(xt_p2n, adst_p2n, src_nx, dst_nx, Hf, half_sp,
                             _num_slices(half_sp, Hf, _ACC_BUDGET["p2n"]))
    sp_p, xt_n2i, adst_i2n = _dense_block_tc(
        acc_p2n, sp_x_p, pn, Hf, wsrc=pi["We"][If_:], sw_next=128,
        wdst=pj["We"][:Nf], be=pj["be"])

    acc_n2i = _sc_edge_phase(xt_n2i, adst_n2i, src_se, dst_se, Nf, half_e,
                             _num_slices(half_e, Nf, _ACC_BUDGET["n2i"]))
    evt_p, xt_i2n = _dense_block_tc(
        acc_n2i, evt_x_p, pi, Nf, wsrc=pj["We"][Nf:], sw_next=128)

    acc_i2n = _sc_edge_phase(xt_i2n, adst_i2n, src_es, dst_es, If_, half_sp,
                             _num_slices(half_sp, If_, _ACC_BUDGET["i2n"]))
    sp2_p, xt_n2p = _dense_block_tc(
        acc_i2n, sp_p, pj, If_, wsrc=pq["We"][Hf:], sw_next=128)

    acc_n2p = _sc_edge_phase(xt_n2p, adst_n2p, src_np, dst_np, Nf, half_h,
                             _num_slices(half_h, Nf, _ACC_BUDGET["n2p"]))
    h2_p, of_p, ox_p = _final_tc(acc_n2p, h_p, _pad_rows(h_of, P_h),
                                 _pad_rows(h_ox, P_h), pq,
                                 params["beta"], params["coord"], Nf)

    return (h2_p[:Nh], sp2_p[:Nsp], evt_p[:Ne], of_p[:Nh], ox_p[:Nh])


# speedup vs baseline: 4.6559x; 1.0630x over previous
import jax
import jax.numpy as jnp
from jax import lax
from jax.experimental import pallas as pl
from jax.experimental.pallas import tpu as pltpu
from jax.experimental.pallas import tpu_sc as plsc

_NS = 16
_TILE = 256
_ACC_BUDGET = {"plane": 3_300_000, "p2n": 5_300_000, "n2i": 1_400_000,
               "i2n": 2_700_000, "n2p": 3_300_000}
_EBUF = 8192


def _round_up(x, m):
    return -(-x // m) * m


def _half_rows(n):
    return _round_up(-(-n // 2), 2560)


def _num_slices(half_p, S, budget):
    k = 1
    while not (half_p % k == 0 and (half_p // k) % 16 == 0
               and (half_p // k + 16) * 2 * S * 4 <= budget):
        k += 1
    return k


def _sc_edge_phase(xt, adst, src_e, dst_e, S, half_p, n_slices):
    n_src_pad, SW = xt.shape
    E_pad = src_e.shape[0]
    E_per = E_pad // _NS
    slice_rows = half_p // n_slices
    assert slice_rows + 16 < (1 << 14) and n_src_pad < (1 << 16)
    zrows = 64 if slice_rows % 64 == 0 else 16
    nzc = slice_rows // zrows
    nzi = -(-nzc // _NS)
    n_fb = S // 16
    C2 = 2 * S
    n_acc = C2 // 128
    ebuf_n = min(_EBUF, E_per)
    n_eb = E_per // ebuf_n
    e_rem = E_per - n_eb * ebuf_n

    mesh = plsc.VectorSubcoreMesh(core_axis_name="c", subcore_axis_name="s",
                                  num_cores=2, num_subcores=_NS)

    def body(x_hbm, adst_hbm, src_hbm, dst_hbm, *rest):
        outs_hbm = rest[:n_acc]
        (cpk_v, ebuf_d, ebuf_s, adsl_v, zbuf, didxb) = rest[n_acc:n_acc + 6]
        k = n_acc + 6
        rowbufs = rest[k:k + 2]
        k += 2
        sbufs = [rest[k:k + n_acc], rest[k + n_acc:k + 2 * n_acc]]
        k += 2 * n_acc
        accs = rest[k:k + n_acc]
        k += n_acc
        gsems = rest[k:k + 2]
        k += 2
        ssems = [rest[k:k + n_acc], rest[k + n_acc:k + 2 * n_acc]]
        cid = lax.axis_index("c")
        sid = lax.axis_index("s")
        ebase = sid * E_per
        zeros_f = jnp.zeros((16,), jnp.float32)
        for r in range(zrows):
            for fb in range(8):
                zbuf[r, pl.ds(fb * 16, 16)] = zeros_f

        def slice_body(sl, _):
            lo = cid * half_p + sl * slice_rows
            hi = lo + slice_rows
            pltpu.sync_copy(adst_hbm.at[pl.ds(lo, slice_rows)],
                            adsl_v.at[pl.ds(0, slice_rows)])
            adsl_v[pl.ds(slice_rows, 16)] = zeros_f

            def zero_chunk(i, carry):
                c = sid + i * _NS

                @pl.when(c < nzc)
                def _():
                    for acc in accs:
                        pltpu.sync_copy(zbuf, acc.at[pl.ds(c * zrows, zrows)])
                return carry

            lax.fori_loop(0, nzi, zero_chunk, 0)
            plsc.subcore_barrier()

            def compact_block(bbase, bsize, off):
                pltpu.sync_copy(dst_hbm.at[pl.ds(ebase + bbase, bsize)],
                                ebuf_d.at[pl.ds(0, bsize)])
                pltpu.sync_copy(src_hbm.at[pl.ds(ebase + bbase, bsize)],
                                ebuf_s.at[pl.ds(0, bsize)])

                def compact(ch, o):
                    d = ebuf_d[pl.ds(ch * 16, 16)]
                    s = ebuf_s[pl.ds(ch * 16, 16)]
                    m = (d >= lo) & (d < hi)
                    cs = plsc.cumsum(jnp.where(m, 1, 0))
                    pos = jnp.where(m, o + cs - 1, E_per + 15)
                    pk = (s << 14) | (d - lo)
                    plsc.store_scatter(cpk_v, [pos], pk)
                    return o + cs[15]

                return lax.fori_loop(0, bsize // 16, compact, off)

            def blk(b, off):
                return compact_block(b * ebuf_n, ebuf_n, off)

            off = lax.fori_loop(0, n_eb, blk, jnp.int32(0))
            if e_rem:
                off = compact_block(n_eb * ebuf_n, e_rem, off)
            cpk_v[pl.ds(off, 16)] = jnp.full((16,), slice_rows, jnp.int32)
            nch = (off + 15) // 16
            iota = lax.iota(jnp.int32, 16)
            col_b = jnp.full((16,), S, jnp.int32)

            def sidx_of(j):
                return lax.shift_right_logical(cpk_v[pl.ds(j * 16, 16)], 14)

            def issue_gather(j, par):
                pltpu.async_copy(x_hbm.at[sidx_of(j)], rowbufs[par],
                                 gsems[par])

            def wait_gather(j, par):
                pltpu.make_async_copy(x_hbm.at[sidx_of(j)], rowbufs[par],
                                      gsems[par]).wait()

            def wait_scatter(par):
                for a in range(n_acc):
                    pltpu.make_async_copy(
                        x_hbm.at[pl.ds(0, 16), pl.ds(0, 128)],
                        sbufs[par][a], ssems[par][a]).wait()

            def compute_issue(j, par):
                pk = cpk_v[pl.ds(j * 16, 16)]
                didx = pk & ((1 << 14) - 1)
                didxb[par, pl.ds(0, 16)] = didx
                avals = plsc.load_gather(adsl_v, [didx])
                bvals = plsc.load_gather(rowbufs[par], [iota, col_b])
                attn = 1.0 / (1.0 + jnp.exp(-(avals + bvals)))
                for e in range(16):
                    ae = attn[e]
                    for fb in range(n_fb):
                        v = rowbufs[par][e, pl.ds(fb * 16, 16)]
                        msg = v * ae
                        p = jnp.exp(msg)
                        cp = fb * 16
                        cq = S + fb * 16
                        sbufs[par][cp // 128][e, pl.ds(cp % 128, 16)] = p
                        sbufs[par][cq // 128][e, pl.ds(cq % 128, 16)] = p * msg
                for a in range(n_acc):
                    pltpu.async_copy(sbufs[par][a],
                                     accs[a].at[didxb.at[par]],
                                     ssems[par][a], add=True)

            @pl.when(nch > 0)
            def _():
                issue_gather(0, 0)

            def proc2(j2, carry):
                j0 = j2 * 2
                j1 = j0 + 1
                wait_gather(j0, 0)

                @pl.when(j1 < nch)
                def _():
                    issue_gather(j1, 1)

                @pl.when(j2 > 0)
                def _():
                    wait_scatter(0)

                compute_issue(j0, 0)

                @pl.when(j1 < nch)
                def _():
                    wait_gather(j1, 1)

                    @pl.when(j1 + 1 < nch)
                    def _():
                        issue_gather(j1 + 1, 0)

                    @pl.when(j2 > 0)
                    def _():
                        wait_scatter(1)

                    compute_issue(j1, 1)
                return carry

            lax.fori_loop(0, (nch + 1) // 2, proc2, 0)

            @pl.when(nch > 0)
            def _():
                wait_scatter(0)

            @pl.when(nch > 1)
            def _():
                wait_scatter(1)

            plsc.subcore_barrier()

            def writeback(i, carry):
                c = sid + i * _NS

                @pl.when(c < nzc)
                def _():
                    for k in range(n_acc):
                        pltpu.sync_copy(
                            accs[k].at[pl.ds(c * zrows, zrows)],
                            outs_hbm[k].at[pl.ds(lo + c * zrows, zrows)])
                return carry

            lax.fori_loop(0, nzi, writeback, 0)
            plsc.subcore_barrier()
            return 0

        lax.fori_loop(0, n_slices, slice_body, 0)

    f = pl.kernel(
        body,
        out_type=tuple(jax.ShapeDtypeStruct((2 * half_p, 128), jnp.float32)
                       for _ in range(n_acc)),
        mesh=mesh,
        compiler_params=pltpu.CompilerParams(needs_layout_passes=False),
        scratch_types=(
            [pltpu.VMEM((E_per + 16,), jnp.int32),
             pltpu.VMEM((ebuf_n,), jnp.int32),
             pltpu.VMEM((ebuf_n,), jnp.int32),
             pltpu.VMEM((slice_rows + 16,), jnp.float32),
             pltpu.VMEM((zrows, 128), jnp.float32),
             pltpu.VMEM((2, 16), jnp.int32)]
            + [pltpu.VMEM((16, SW), jnp.float32) for _ in range(2)]
            + [pltpu.VMEM((16, 128), jnp.float32)
               for _ in range(2 * n_acc)]
            + [pltpu.VMEM_SHARED((slice_rows + 16, 128), jnp.float32)
               for _ in range(n_acc)]
            + [pltpu.SemaphoreType.DMA for _ in range(2)]
            + [pltpu.SemaphoreType.DMA for _ in range(2 * n_acc)]
        ),
    )
    res = f(xt, adst, src_e, dst_e)
    return list(res) if isinstance(res, (tuple, list)) else [res]


def _mish(x):
    return x * jnp.tanh(jax.nn.softplus(x))


def _row_spec(c):
    return pl.BlockSpec((_TILE, c), lambda i: (i, 0))


def _full_spec(a):
    return pl.BlockSpec(a.shape, lambda i: (0,) * a.ndim)


def _table_tc(x, wsrc, sw):
    P, F = x.shape
    grid = (P // _TILE,)

    def body(x_ref, w_ref, o_ref):
        xv = x_ref[...]
        b = jnp.dot(xv, w_ref[...], preferred_element_type=jnp.float32)
        o_ref[...] = jnp.concatenate(
            [xv, b, jnp.zeros((_TILE, sw - F - 1), jnp.float32)], axis=1)

    return pl.pallas_call(
        body, grid=grid,
        out_shape=jax.ShapeDtypeStruct((P, sw), jnp.float32),
        in_specs=[_row_spec(F), _full_spec(wsrc)],
        out_specs=_row_spec(sw))(x, wsrc)


def _scalars_tc(x, cols):
    P, F = x.shape
    grid = (P // _TILE,)

    def body(x_ref, *refs):
        n = len(cols)
        wrefs = refs[:n]
        brefs = {i: r for i, r in zip(
            [i for i, c in enumerate(cols) if c[1] is not None],
            refs[n:n + sum(c[1] is not None for c in cols)])}
        orefs = refs[n + len(brefs):]
        xv = x_ref[...]
        for i in range(n):
            v = jnp.dot(xv, wrefs[i][...], preferred_element_type=jnp.float32)
            if i in brefs:
                v = v + brefs[i][...]
            orefs[i][...] = v

    args = [x] + [c[0] for c in cols] + [c[1].reshape(1, 1) for c in cols
                                         if c[1] is not None]
    in_specs = [_row_spec(F)] + [_full_spec(a) for a in args[1:]]
    outs = [jax.ShapeDtypeStruct((P, 1), jnp.float32) for _ in cols]
    out_specs = [_row_spec(1) for _ in cols]
    res = pl.pallas_call(body, grid=grid, out_shape=outs,
                         in_specs=in_specs, out_specs=out_specs)(*args)
    return [r.reshape(-1) for r in res]


def _aggr_from(acc_refs, S):
    if len(acc_refs) == 2:
        sump = acc_refs[0][...]
        sumpm = acc_refs[1][...]
    else:
        a = acc_refs[0][...]
        sump, sumpm = a[:, :S], a[:, S:]
    return sumpm / (sump + 1e-16)


def _dense_block_tc(accs, xdst, p, S, wsrc, sw_next, wdst=None, be=None):
    n_acc = len(accs)
    P = accs[0].shape[0]
    T = xdst.shape[1]
    O = p["W2"].shape[0]
    grid = (P // _TILE,)
    W1a, W1b = p["W1"][:S], p["W1"][S:]
    b1 = p["b1"].reshape(1, O)
    b2 = p["b2"].reshape(1, O)
    have_dst = wdst is not None
    outs = [jax.ShapeDtypeStruct((P, O), jnp.float32),
            jax.ShapeDtypeStruct((P, sw_next), jnp.float32)]
    out_specs = [_row_spec(O), _row_spec(sw_next)]
    if have_dst:
        outs.append(jax.ShapeDtypeStruct((P, 1), jnp.float32))
        out_specs.append(_row_spec(1))

    def body(*allrefs):
        acc_refs = allrefs[:n_acc]
        (x_ref, W1a_ref, W1b_ref, b1_ref, W2_ref, b2_ref,
         wsrc_ref) = allrefs[n_acc:n_acc + 7]
        refs = allrefs[n_acc + 7:]
        i = 0
        if have_dst:
            wdst_ref, be_ref = refs[0], refs[1]
            i = 2
        h_ref, xt_ref = refs[i], refs[i + 1]
        adst_ref = refs[i + 2] if have_dst else None
        aggr = _aggr_from(acc_refs, S)
        h1 = _mish(jnp.dot(aggr, W1a_ref[...],
                           preferred_element_type=jnp.float32)
                   + jnp.dot(x_ref[...], W1b_ref[...],
                             preferred_element_type=jnp.float32)
                   + b1_ref[...])
        h = _mish(jnp.dot(h1, W2_ref[...],
                          preferred_element_type=jnp.float32) + b2_ref[...])
        h_ref[...] = h
        bsrc = jnp.dot(h, wsrc_ref[...], preferred_element_type=jnp.float32)
        xt_ref[...] = jnp.concatenate(
            [h, bsrc, jnp.zeros((_TILE, sw_next - O - 1), jnp.float32)],
            axis=1)
        if have_dst:
            adst_ref[...] = jnp.dot(h, wdst_ref[...],
                                    preferred_element_type=jnp.float32) + be_ref[...]

    args = list(accs) + [xdst, W1a, W1b, b1, p["W2"], b2, wsrc]
    if have_dst:
        args += [wdst, be.reshape(1, 1)]
    in_specs = ([_row_spec(128)] * n_acc + [_row_spec(T)]
                + [_full_spec(a) for a in args[n_acc + 1:]])
    res = list(pl.pallas_call(body, grid=grid, out_shape=outs,
                              in_specs=in_specs, out_specs=out_specs)(*args))
    if have_dst:
        res[2] = res[2].reshape(-1)
    return res


def _final_tc(accs, hdst, h_of, h_ox, p, beta, coord, S):
    n_acc = len(accs)
    P = accs[0].shape[0]
    T = hdst.shape[1]
    O = p["W2"].shape[0]
    Inst = h_ox.shape[1]
    grid = (P // _TILE,)
    W1a, W1b = p["W1"][:S], p["W1"][S:]
    b1 = p["b1"].reshape(1, O)
    b2 = p["b2"].reshape(1, O)
    (Wb1, bb1), (Wb2, bb2), (Wb3, bb3) = beta
    (Wc1, bc1), (Wc2, bc2), (Wc3, bc3) = coord
    Wb1a, Wb1b = Wb1[:1], Wb1[1:]
    Wc1a, Wc1b = Wc1[:Inst], Wc1[Inst:]
    hidden = Wb2.shape[0]

    def body(*allrefs):
        acc_refs = allrefs[:n_acc]
        (x_ref, of_in_ref, ox_in_ref,
         W1a_ref, W1b_ref, b1_ref, W2_ref, b2_ref,
         Wb1a_ref, Wb1b_ref, bb1_ref, Wb2_ref, bb2_ref, Wb3_ref, bb3_ref,
         Wc1a_ref, Wc1b_ref, bc1_ref, Wc2_ref, bc2_ref, Wc3_ref, bc3_ref,
         h_ref, of_ref, ox_ref) = allrefs[n_acc:]
        aggr = _aggr_from(acc_refs, S)
        h1 = _mish(jnp.dot(aggr, W1a_ref[...],
                           preferred_element_type=jnp.float32)
                   + jnp.dot(x_ref[...], W1b_ref[...],
                             preferred_element_type=jnp.float32)
                   + b1_ref[...])
        h = _mish(jnp.dot(h1, W2_ref[...],
                          preferred_element_type=jnp.float32) + b2_ref[...])
        h_ref[...] = h
        u = _mish(jnp.dot(of_in_ref[...], Wb1a_ref[...],
                          preferred_element_type=jnp.float32)
                  + jnp.dot(h, Wb1b_ref[...],
                            preferred_element_type=jnp.float32)
                  + bb1_ref[...])
        u = _mish(jnp.dot(u, Wb2_ref[...],
                          preferred_element_type=jnp.float32) + bb2_ref[...])
        of_ref[...] = jax.nn.sigmoid(
            jnp.dot(u, Wb3_ref[...], preferred_element_type=jnp.float32)
            + bb3_ref[...])
        v = _mish(jnp.dot(ox_in_ref[...], Wc1a_ref[...],
                          preferred_element_type=jnp.float32)
                  + jnp.dot(h, Wc1b_ref[...],
                            preferred_element_type=jnp.float32)
                  + bc1_ref[...])
        v = _mish(jnp.dot(v, Wc2_ref[...],
                          preferred_element_type=jnp.float32) + bc2_ref[...])
        ox_ref[...] = jnp.dot(v, Wc3_ref[...],
                              preferred_element_type=jnp.float32) + bc3_ref[...]

    args = list(accs) + [hdst, h_of, h_ox, W1a, W1b, b1, p["W2"], b2,
                         Wb1a, Wb1b, bb1.reshape(1, hidden),
                         Wb2, bb2.reshape(1, hidden), Wb3, bb3.reshape(1, 1),
                         Wc1a, Wc1b, bc1.reshape(1, hidden),
                         Wc2, bc2.reshape(1, hidden), Wc3,
                         bc3.reshape(1, Inst)]
    in_specs = ([_row_spec(128)] * n_acc
                + [_row_spec(T), _row_spec(1), _row_spec(Inst)]
                + [_full_spec(a) for a in args[n_acc + 3:]])
    outs = [jax.ShapeDtypeStruct((P, O), jnp.float32),
            jax.ShapeDtypeStruct((P, 1), jnp.float32),
            jax.ShapeDtypeStruct((P, Inst), jnp.float32)]
    out_specs = [_row_spec(O), _row_spec(1), _row_spec(Inst)]
    return pl.pallas_call(body, grid=grid, out_shape=outs,
                          in_specs=in_specs, out_specs=out_specs)(*args)


def _pad_rows(x, P, C=None):
    C = C if C is not None else x.shape[1]
    return jnp.pad(x, ((0, P - x.shape[0]), (0, C - x.shape[1])))


def _pad_edges(src, dst):
    E = src.shape[0]
    E_pad = _round_up(E, 256)
    src_p = jnp.pad(src, (0, E_pad - E))
    dst_p = jnp.pad(dst, (0, E_pad - E), constant_values=-1)
    return src_p, dst_p


def kernel(h_x, sp_x, evt_x, h_of, h_ox, planar_edge_index, nexus_src,
           nexus_dst, sp_evt_src, sp_evt_dst, params):
    Nh, Hf = h_x.shape
    Nsp, Nf = sp_x.shape
    Ne, If_ = evt_x.shape
    half_h, half_sp, half_e = _half_rows(Nh), _half_rows(Nsp), _half_rows(Ne)
    P_h, P_sp, P_e = 2 * half_h, 2 * half_sp, 2 * half_e

    pp = params["plane"]
    pn = params["p2n"]
    pi = params["n2i"]
    pj = params["i2n"]
    pq = params["n2p"]

    h_x_p = _pad_rows(h_x, P_h)
    sp_x_p = _pad_rows(sp_x, P_sp)
    evt_x_p = _pad_rows(evt_x, P_e)

    src_pl, dst_pl = _pad_edges(planar_edge_index[0], planar_edge_index[1])
    src_nx, dst_nx = _pad_edges(nexus_src, nexus_dst)
    src_se, dst_se = _pad_edges(sp_evt_src, sp_evt_dst)
    src_es, dst_es = _pad_edges(sp_evt_dst, sp_evt_src)
    src_np, dst_np = _pad_edges(nexus_dst, nexus_src)

    xt_pl = _table_tc(h_x_p, pp["We"][Hf:], 256)
    (adst_pl,) = _scalars_tc(h_x_p, [(pp["We"][:Hf], pp["be"])])
    (adst_p2n,) = _scalars_tc(sp_x_p, [(pn["We"][:Nf], pn["be"])])
    (adst_n2i,) = _scalars_tc(evt_x_p, [(pi["We"][:If_], pi["be"])])

    acc_pl = _sc_edge_phase(xt_pl, adst_pl, src_pl, dst_pl, Hf, half_h,
                            _num_slices(half_h, Hf, _ACC_BUDGET["plane"]))
    h_p, xt_p2n, adst_n2p = _dense_block_tc(
        acc_pl, h_x_p, pp, Hf, wsrc=pn["We"][Nf:], sw_next=256,
        wdst=pq["We"][:Hf], be=pq["be"])

    acc_p2n = _sc_edge_phase(xt_p2n, adst_p2n, src_nx, dst_nx, Hf, half_sp,
                             _num_slices(half_sp, Hf, _ACC_BUDGET["p2n"]))
    sp_p, xt_n2i, adst_i2n = _dense_block_tc(
        acc_p2n, sp_x_p, pn, Hf, wsrc=pi["We"][If_:], sw_next=128,
        wdst=pj["We"][:Nf], be=pj["be"])

    acc_n2i = _sc_edge_phase(xt_n2i, adst_n2i, src_se, dst_se, Nf, half_e,
                             _num_slices(half_e, Nf, _ACC_BUDGET["n2i"]))
    evt_p, xt_i2n = _dense_block_tc(
        acc_n2i, evt_x_p, pi, Nf, wsrc=pj["We"][Nf:], sw_next=128)

    acc_i2n = _sc_edge_phase(xt_i2n, adst_i2n, src_es, dst_es, If_, half_sp,
                             _num_slices(half_sp, If_, _ACC_BUDGET["i2n"]))
    sp2_p, xt_n2p = _dense_block_tc(
        acc_i2n, sp_p, pj, If_, wsrc=pq["We"][Hf:], sw_next=128)

    acc_n2p = _sc_edge_phase(xt_n2p, adst_n2p, src_np, dst_np, Nf, half_h,
                             _num_slices(half_h, Nf, _ACC_BUDGET["n2p"]))
    h2_p, of_p, ox_p = _final_tc(acc_n2p, h_p, _pad_rows(h_of, P_h),
                                 _pad_rows(h_ox, P_h), pq,
                                 params["beta"], params["coord"], Nf)

    return (h2_p[:Nh], sp2_p[:Nsp], evt_p[:Ne], of_p[:Nh], ox_p[:Nh])
